# baseline (speedup 1.0000x reference)
import jax
import jax.numpy as jnp
from jax import lax
from jax.experimental import pallas as pl
from jax.experimental.pallas import tpu as pltpu

KCLS = 128
BATCH = 64
PIX = 32 * 32


def _onehot_body(x_ref, o_ref):
    x = x_ref[0]
    cls = lax.broadcasted_iota(jnp.int32, o_ref.shape, 1)
    o_ref[...] = (x[:, None, :] == cls).astype(jnp.float32)


def kernel(x):
    bb = 16
    x = x.astype(jnp.int32).reshape(BATCH // bb, bb, PIX)
    out = pl.pallas_call(
        _onehot_body,
        grid=(BATCH // bb,),
        in_specs=[pl.BlockSpec((1, bb, PIX), lambda i: (i, 0, 0))],
        out_specs=pl.BlockSpec((bb, KCLS, PIX), lambda i: (i, 0, 0)),
        out_shape=jax.ShapeDtypeStruct((BATCH, KCLS, PIX), jnp.float32),
        compiler_params=pltpu.CompilerParams(
            dimension_semantics=("arbitrary",),
        ),
    )(x)
    return out.reshape(BATCH, KCLS, 32, 32)
